# Initial kernel scaffold; baseline (speedup 1.0000x reference)
import jax
import jax.numpy as jnp
from jax.experimental import pallas as pl


def kernel(instance, compsrc):
    raise NotImplementedError("write your pallas kernel here")


# speedup vs baseline: 5.0302x; 5.0302x over previous
import functools

import jax
import jax.numpy as jnp
from jax import lax
from jax.experimental import pallas as pl
from jax.experimental.pallas import tpu as pltpu
from jax.experimental.pallas import tpu_sc as plsc

MAXINS = 50
D = 36
DP = 40
BATCHES = 4
PIX = 192 * 640
ROWS = BATCHES * PIX
ROWS_PER_CORE = ROWS // 2
ROWS_PER_TILE = ROWS_PER_CORE // 16
CHUNK = 1024
NCHUNK = ROWS_PER_TILE // CHUNK
GRP = 128
SEG_STRIDE = 64
NSEG = BATCHES * SEG_STRIDE
TD = 128
CBLK = 2048
BLKS_PER_BATCH = PIX // CBLK


def _compress_body(ids_ref, src_ref, out_ref):
    g = pl.program_id(0)
    ids = ids_ref[0, 0, :]
    seg0 = SEG_STRIDE * (g // BLKS_PER_BATCH)
    onehot = (ids[:, None]
              == lax.broadcasted_iota(jnp.int32, (CBLK, SEG_STRIDE), 1) + seg0
              ).astype(jnp.float32)
    partial = lax.dot_general(
        onehot, src_ref[0],
        dimension_numbers=(((0,), (0,)), ((), ())),
        preferred_element_type=jnp.float32,
        precision=lax.Precision.HIGHEST)

    @pl.when(g % BLKS_PER_BATCH == 0)
    def _init():
        out_ref[...] = jnp.zeros_like(out_ref)

    out_ref[:, :D] += partial


def _inflate_body(ids_hbm, table_hbm, out_hbm, t40_hbm, ids_c, rows_v, row_v):
    c = lax.axis_index("c")
    s = lax.axis_index("s")
    row_base = c * ROWS_PER_CORE + s * ROWS_PER_TILE

    def pack_row(r, carry):
        pltpu.sync_copy(table_hbm.at[r], row_v)
        pltpu.sync_copy(row_v.at[pl.ds(0, DP)], t40_hbm.at[r])
        return carry
    r0 = c * (2 * SEG_STRIDE) + s * 8
    lax.fori_loop(r0, r0 + 8, pack_row, 0)

    plsc.subcore_barrier()

    def inf_chunk(ch, carry):
        base = row_base + ch * CHUNK
        for j in range(CHUNK // GRP):
            pltpu.sync_copy(ids_hbm.at[pl.ds(base + j * GRP, GRP)], ids_c)
            pltpu.sync_copy(t40_hbm.at[ids_c],
                            rows_v.at[pl.ds(j * GRP, GRP)])
        pltpu.sync_copy(rows_v, out_hbm.at[pl.ds(base, CHUNK)])
        return carry
    lax.fori_loop(0, NCHUNK, inf_chunk, 0)


@jax.jit
def _run(ids_off, src2):
    ids3 = ids_off.reshape(ROWS // CBLK, 1, CBLK)
    src3 = src2.reshape(ROWS // CBLK, CBLK, D)
    table = pl.pallas_call(
        _compress_body,
        grid=(ROWS // CBLK,),
        in_specs=[
            pl.BlockSpec((1, 1, CBLK), lambda g: (g, 0, 0)),
            pl.BlockSpec((1, CBLK, D), lambda g: (g, 0, 0)),
        ],
        out_specs=pl.BlockSpec((SEG_STRIDE, TD),
                               lambda g: (g // BLKS_PER_BATCH, 0)),
        out_shape=jax.ShapeDtypeStruct((NSEG, TD), jnp.float32),
    )(ids3, src3)

    mesh = plsc.VectorSubcoreMesh(
        core_axis_name="c", subcore_axis_name="s",
        num_cores=2, num_subcores=16)
    inflate = functools.partial(
        pl.kernel,
        out_type=(jax.ShapeDtypeStruct((ROWS, DP), jnp.float32),
                  jax.ShapeDtypeStruct((NSEG, DP), jnp.float32)),
        mesh=mesh,
        scratch_types=[
            pltpu.VMEM((GRP,), jnp.int32),
            pltpu.VMEM((CHUNK, DP), jnp.float32),
            pltpu.VMEM((TD,), jnp.float32),
        ],
        compiler_params=pltpu.CompilerParams(use_tc_tiling_on_sc=False),
    )(_inflate_body)
    out40, _ = inflate(ids_off, table)
    return out40[:, :D]


def kernel(instance, compsrc):
    ids_off = (instance.reshape(BATCHES, PIX)
               + SEG_STRIDE * jnp.arange(BATCHES, dtype=jnp.int32)[:, None]
               ).reshape(ROWS)
    src2 = compsrc.reshape(ROWS, D)
    out = _run(ids_off, src2)
    return out.reshape(BATCHES, 192, 640, 6, 6)


# speedup vs baseline: 5.2699x; 1.0477x over previous
import functools

import jax
import jax.numpy as jnp
from jax import lax
from jax.experimental import pallas as pl
from jax.experimental.pallas import tpu as pltpu
from jax.experimental.pallas import tpu_sc as plsc

MAXINS = 50
D = 36
DP = 40
BATCHES = 4
PIX = 192 * 640
ROWS = BATCHES * PIX
ROWS_PER_CORE = ROWS // 2
ROWS_PER_TILE = ROWS_PER_CORE // 16
CHUNK = 1024
NCHUNK = ROWS_PER_TILE // CHUNK
GRP = 128
SEG_STRIDE = 64
NSEG = BATCHES * SEG_STRIDE
TD = 128
CBLK = 4096
BLKS_PER_BATCH = PIX // CBLK


def _compress_body(ids_ref, src_ref, out_ref):
    g = pl.program_id(0)
    ids = ids_ref[0, 0, :]
    seg0 = SEG_STRIDE * (g // BLKS_PER_BATCH)
    onehot = (ids[:, None]
              == lax.broadcasted_iota(jnp.int32, (CBLK, SEG_STRIDE), 1) + seg0
              ).astype(jnp.float32)
    partial = lax.dot_general(
        onehot, src_ref[0],
        dimension_numbers=(((0,), (0,)), ((), ())),
        preferred_element_type=jnp.float32,
        precision=lax.Precision.HIGHEST)

    @pl.when(g % BLKS_PER_BATCH == 0)
    def _init():
        out_ref[...] = jnp.zeros_like(out_ref)

    out_ref[:, :D] += partial


def _inflate_body(ids_hbm, table_hbm, out_hbm, t40_hbm, ids_c, rows_v, row_v):
    c = lax.axis_index("c")
    s = lax.axis_index("s")
    row_base = c * ROWS_PER_CORE + s * ROWS_PER_TILE

    def pack_row(r, carry):
        pltpu.sync_copy(table_hbm.at[r], row_v)
        pltpu.sync_copy(row_v.at[pl.ds(0, DP)], t40_hbm.at[r])
        return carry
    r0 = c * (2 * SEG_STRIDE) + s * 8
    lax.fori_loop(r0, r0 + 8, pack_row, 0)

    plsc.subcore_barrier()

    def inf_chunk(ch, carry):
        base = row_base + ch * CHUNK
        for j in range(CHUNK // GRP):
            pltpu.sync_copy(ids_hbm.at[pl.ds(base + j * GRP, GRP)], ids_c)
            pltpu.sync_copy(t40_hbm.at[ids_c],
                            rows_v.at[pl.ds(j * GRP, GRP)])
        pltpu.sync_copy(rows_v, out_hbm.at[pl.ds(base, CHUNK)])
        return carry
    lax.fori_loop(0, NCHUNK, inf_chunk, 0)


@jax.jit
def _run(ids_off, src2):
    ids3 = ids_off.reshape(ROWS // CBLK, 1, CBLK)
    src3 = src2.reshape(ROWS // CBLK, CBLK, D)
    table = pl.pallas_call(
        _compress_body,
        grid=(ROWS // CBLK,),
        in_specs=[
            pl.BlockSpec((1, 1, CBLK), lambda g: (g, 0, 0)),
            pl.BlockSpec((1, CBLK, D), lambda g: (g, 0, 0)),
        ],
        out_specs=pl.BlockSpec((SEG_STRIDE, TD),
                               lambda g: (g // BLKS_PER_BATCH, 0)),
        out_shape=jax.ShapeDtypeStruct((NSEG, TD), jnp.float32),
    )(ids3, src3)

    mesh = plsc.VectorSubcoreMesh(
        core_axis_name="c", subcore_axis_name="s",
        num_cores=2, num_subcores=16)
    inflate = functools.partial(
        pl.kernel,
        out_type=(jax.ShapeDtypeStruct((ROWS, DP), jnp.float32),
                  jax.ShapeDtypeStruct((NSEG, DP), jnp.float32)),
        mesh=mesh,
        scratch_types=[
            pltpu.VMEM((GRP,), jnp.int32),
            pltpu.VMEM((CHUNK, DP), jnp.float32),
            pltpu.VMEM((TD,), jnp.float32),
        ],
        compiler_params=pltpu.CompilerParams(use_tc_tiling_on_sc=False),
    )(_inflate_body)
    out40, _ = inflate(ids_off, table)
    return out40[:, :D]


def kernel(instance, compsrc):
    ids_off = (instance.reshape(BATCHES, PIX)
               + SEG_STRIDE * jnp.arange(BATCHES, dtype=jnp.int32)[:, None]
               ).reshape(ROWS)
    src2 = compsrc.reshape(ROWS, D)
    out = _run(ids_off, src2)
    return out.reshape(BATCHES, 192, 640, 6, 6)


# speedup vs baseline: 5.3612x; 1.0173x over previous
import functools

import jax
import jax.numpy as jnp
from jax import lax
from jax.experimental import pallas as pl
from jax.experimental.pallas import tpu as pltpu
from jax.experimental.pallas import tpu_sc as plsc

MAXINS = 50
D = 36
DP = 40
BATCHES = 4
PIX = 192 * 640
ROWS = BATCHES * PIX
ROWS_PER_CORE = ROWS // 2
ROWS_PER_TILE = ROWS_PER_CORE // 16
CHUNK = 1024
NCHUNK = ROWS_PER_TILE // CHUNK
GRP = 128
SEG_STRIDE = 64
NSEG = BATCHES * SEG_STRIDE
TD = 128
CBLK = 4096
BLKS_PER_BATCH = PIX // CBLK


def _compress_body(ids_ref, src_ref, out_ref):
    g = pl.program_id(0)
    ids = ids_ref[0, 0, :]
    seg0 = SEG_STRIDE * (g // BLKS_PER_BATCH)
    onehot = (ids[:, None]
              == lax.broadcasted_iota(jnp.int32, (CBLK, SEG_STRIDE), 1) + seg0
              ).astype(jnp.float32)
    partial = lax.dot_general(
        onehot, src_ref[0],
        dimension_numbers=(((0,), (0,)), ((), ())),
        preferred_element_type=jnp.float32,
        precision=lax.Precision.HIGHEST)

    @pl.when(g % BLKS_PER_BATCH == 0)
    def _init():
        out_ref[...] = jnp.zeros_like(out_ref)

    out_ref[:, :D] += partial


def _inflate_body(ids_hbm, table_hbm, out_hbm, t40_hbm, ids8, rows_v, row_v,
                  sem):
    c = lax.axis_index("c")
    s = lax.axis_index("s")
    row_base = c * ROWS_PER_CORE + s * ROWS_PER_TILE

    def pack_row(r, carry):
        pltpu.sync_copy(table_hbm.at[r], row_v)
        pltpu.sync_copy(row_v.at[pl.ds(0, DP)], t40_hbm.at[r])
        return carry
    r0 = c * (2 * SEG_STRIDE) + s * 8
    lax.fori_loop(r0, r0 + 8, pack_row, 0)

    plsc.subcore_barrier()

    def inf_chunk(ch, carry):
        base = row_base + ch * CHUNK
        pltpu.sync_copy(ids_hbm.at[pl.ds(base // GRP, CHUNK // GRP)], ids8)
        descs = [
            pltpu.async_copy(t40_hbm.at[ids8.at[j]],
                             rows_v.at[pl.ds(j * GRP, GRP)], sem)
            for j in range(CHUNK // GRP)
        ]
        for d in descs:
            d.wait()
        pltpu.sync_copy(rows_v, out_hbm.at[pl.ds(base, CHUNK)])
        return carry
    lax.fori_loop(0, NCHUNK, inf_chunk, 0)


@jax.jit
def _run(ids_off, src2):
    ids3 = ids_off.reshape(ROWS // CBLK, 1, CBLK)
    src3 = src2.reshape(ROWS // CBLK, CBLK, D)
    table = pl.pallas_call(
        _compress_body,
        grid=(ROWS // CBLK,),
        in_specs=[
            pl.BlockSpec((1, 1, CBLK), lambda g: (g, 0, 0)),
            pl.BlockSpec((1, CBLK, D), lambda g: (g, 0, 0)),
        ],
        out_specs=pl.BlockSpec((SEG_STRIDE, TD),
                               lambda g: (g // BLKS_PER_BATCH, 0)),
        out_shape=jax.ShapeDtypeStruct((NSEG, TD), jnp.float32),
    )(ids3, src3)

    mesh = plsc.VectorSubcoreMesh(
        core_axis_name="c", subcore_axis_name="s",
        num_cores=2, num_subcores=16)
    inflate = functools.partial(
        pl.kernel,
        out_type=(jax.ShapeDtypeStruct((ROWS, DP), jnp.float32),
                  jax.ShapeDtypeStruct((NSEG, DP), jnp.float32)),
        mesh=mesh,
        scratch_types=[
            pltpu.VMEM((CHUNK // GRP, GRP), jnp.int32),
            pltpu.VMEM((CHUNK, DP), jnp.float32),
            pltpu.VMEM((TD,), jnp.float32),
            pltpu.SemaphoreType.DMA,
        ],
        compiler_params=pltpu.CompilerParams(use_tc_tiling_on_sc=False),
    )(_inflate_body)
    ids2 = ids_off.reshape(ROWS // GRP, GRP)
    out40, _ = inflate(ids2, table)
    return out40[:, :D]


def kernel(instance, compsrc):
    ids_off = (instance.reshape(BATCHES, PIX)
               + SEG_STRIDE * jnp.arange(BATCHES, dtype=jnp.int32)[:, None]
               ).reshape(ROWS)
    src2 = compsrc.reshape(ROWS, D)
    out = _run(ids_off, src2)
    return out.reshape(BATCHES, 192, 640, 6, 6)
